# Initial kernel scaffold; baseline (speedup 1.0000x reference)
import jax
import jax.numpy as jnp
from jax.experimental import pallas as pl


def kernel(node_features, parent_index, samp_z, samp_x_raw, W1, b1, W2, b2, Wr1, br1, Wr2, br2):
    raise NotImplementedError("write your pallas kernel here")


# speedup vs baseline: 1.3843x; 1.3843x over previous
import math

import jax
import jax.numpy as jnp
from jax import lax
from jax.experimental import pallas as pl
from jax.experimental.pallas import tpu as pltpu

NTIPS = 256
HID = 256
LAT = 50
B = 4
Z = 32
NDIM = 509
NNODE = 510
ZB = 4
LOG2PI = math.log(2.0 * math.pi)


def _elu(x):
    return jnp.where(x > 0, x, jnp.exp(jnp.minimum(x, 0.0)) - 1.0)


def _front_kernel(h_ref, pi_ref, w1_ref, b1_ref, w2_ref, b2_ref,
                  wr1a_ref, br1_ref, base_ref):
    h = h_ref[0]
    x = _elu(jnp.dot(h, w1_ref[...], preferred_element_type=jnp.float32)
             + b1_ref[...])
    ms = _elu(jnp.dot(x, w2_ref[...], preferred_element_type=jnp.float32)
              + b2_ref[...])
    pi = pi_ref[0]
    j = lax.broadcasted_iota(jnp.int32, (NNODE, NDIM), 0)
    pt = (j == pi).astype(jnp.float32)
    parents = lax.dot_general(pt, ms, (((0,), (0,)), ((), ())),
                              preferred_element_type=jnp.float32)
    mst = jnp.maximum(ms[:NDIM, :], parents)
    base_ref[0] = lax.dot_general(
        wr1a_ref[...], mst, (((0,), (1,)), ((), ())),
        preferred_element_type=jnp.float32) + br1_ref[...]


def _readout_kernel(base_ref, sz_ref, wr1b_ref, wr2_ref, br2_ref, sxr_ref,
                    slb_ref, logq_ref, slb_s):
    zb = pl.program_id(1)
    base = base_ref[0]
    logqs = []
    for k in range(ZB):
        sz = sz_ref[0, k]
        zc = lax.dot_general(wr1b_ref[...], sz, (((0,), (1,)), ((), ())),
                             preferred_element_type=jnp.float32)
        r = _elu(base + zc)
        out = lax.dot_general(wr2_ref[...], r, (((0,), (0,)), ((), ())),
                              preferred_element_type=jnp.float32) + br2_ref[...]
        mean = out[0:1, :]
        ls = jnp.maximum(out[1:2, :], -3.0)

        if k == 0:
            @pl.when(zb == 0)
            def _():
                slb_s[...] = sxr_ref[0] * jnp.exp(ls) + mean - 2.0
                slb_ref[0] = slb_s[...]

        slb = slb_s[...]
        dev = (slb - mean + 2.0) * jnp.exp(-ls)
        logq = -0.5 * jnp.sum(LOG2PI + dev * dev) - jnp.sum(ls)
        logqs.append(jnp.full((1, 1, 1, 128), logq, jnp.float32))
    logq_ref[...] = jnp.concatenate(logqs, axis=1)


def kernel(node_features, parent_index, samp_z, samp_x_raw, W1, b1, W2, b2,
           Wr1, br1, Wr2, br2):
    f32 = jnp.float32
    pi = parent_index.astype(jnp.int32).reshape(B, 1, NDIM)
    sxr = samp_x_raw.reshape(B, 1, NDIM)
    b1r = b1.reshape(1, HID)
    b2r = b2.reshape(1, HID)
    wr1a = Wr1[:HID]
    wr1b = Wr1[HID:]
    br1c = br1.reshape(HID, 1)
    br2c = br2.reshape(2, 1)

    base = pl.pallas_call(
        _front_kernel,
        grid=(B,),
        in_specs=[
            pl.BlockSpec((1, NNODE, NTIPS), lambda b: (b, 0, 0)),
            pl.BlockSpec((1, 1, NDIM), lambda b: (b, 0, 0)),
            pl.BlockSpec((NTIPS, HID), lambda b: (0, 0)),
            pl.BlockSpec((1, HID), lambda b: (0, 0)),
            pl.BlockSpec((HID, HID), lambda b: (0, 0)),
            pl.BlockSpec((1, HID), lambda b: (0, 0)),
            pl.BlockSpec((HID, HID), lambda b: (0, 0)),
            pl.BlockSpec((HID, 1), lambda b: (0, 0)),
        ],
        out_specs=pl.BlockSpec((1, HID, NDIM), lambda b: (b, 0, 0)),
        out_shape=jax.ShapeDtypeStruct((B, HID, NDIM), f32),
    )(node_features, pi, W1, b1r, W2, b2r, wr1a, br1c)

    slb_p, logq_p = pl.pallas_call(
        _readout_kernel,
        grid=(B, Z // ZB),
        in_specs=[
            pl.BlockSpec((1, HID, NDIM), lambda b, z: (b, 0, 0)),
            pl.BlockSpec((1, ZB, NDIM, LAT), lambda b, z: (b, z, 0, 0)),
            pl.BlockSpec((LAT, HID), lambda b, z: (0, 0)),
            pl.BlockSpec((HID, 2), lambda b, z: (0, 0)),
            pl.BlockSpec((2, 1), lambda b, z: (0, 0)),
            pl.BlockSpec((1, 1, NDIM), lambda b, z: (b, 0, 0)),
        ],
        out_specs=[
            pl.BlockSpec((1, 1, NDIM), lambda b, z: (b, 0, 0)),
            pl.BlockSpec((1, ZB, 1, 128), lambda b, z: (b, z, 0, 0)),
        ],
        out_shape=[
            jax.ShapeDtypeStruct((B, 1, NDIM), f32),
            jax.ShapeDtypeStruct((B, Z, 1, 128), f32),
        ],
        scratch_shapes=[pltpu.VMEM((1, NDIM), f32)],
    )(base, samp_z, wr1b, Wr2, br2c, sxr)

    samp_log_branch = slb_p[:, 0, :]
    logq_branch_batch = logq_p[:, :, 0, 0]
    return (samp_log_branch, logq_branch_batch)


# speedup vs baseline: 1.4864x; 1.0738x over previous
import math

import jax
import jax.numpy as jnp
from jax import lax
from jax.experimental import pallas as pl
from jax.experimental.pallas import tpu as pltpu

NTIPS = 256
HID = 256
LAT = 50
B = 4
Z = 32
NDIM = 509
NNODE = 510
ZB = 8
LOG2PI = math.log(2.0 * math.pi)


def _elu(x):
    return jnp.where(x > 0, x, jnp.exp(jnp.minimum(x, 0.0)) - 1.0)


def _front_kernel(h_ref, pi_ref, w1_ref, b1_ref, w2_ref, b2_ref,
                  wr1a_ref, br1_ref, base_ref):
    h = h_ref[0]
    x = _elu(jnp.dot(h, w1_ref[...], preferred_element_type=jnp.float32)
             + b1_ref[...])
    ms = _elu(jnp.dot(x, w2_ref[...], preferred_element_type=jnp.float32)
              + b2_ref[...])
    pi = pi_ref[0]
    j = lax.broadcasted_iota(jnp.int32, (NNODE, NDIM), 0)
    pt = (j == pi).astype(jnp.float32)
    parents = lax.dot_general(pt, ms, (((0,), (0,)), ((), ())),
                              preferred_element_type=jnp.float32)
    mst = jnp.maximum(ms[:NDIM, :], parents)
    base_ref[0] = lax.dot_general(
        wr1a_ref[...], mst, (((0,), (1,)), ((), ())),
        preferred_element_type=jnp.float32) + br1_ref[...]


def _readout_kernel(base_ref, sz_ref, wr1b_ref, wr2_ref, br2_ref, sxr_ref,
                    slb_ref, logq_ref, slb_s):
    zb = pl.program_id(1)
    base = base_ref[0]
    logqs = []
    for k in range(ZB):
        sz = sz_ref[0, k]
        zc = lax.dot_general(wr1b_ref[...], sz, (((0,), (1,)), ((), ())),
                             preferred_element_type=jnp.float32)
        r = _elu(base + zc)
        out = lax.dot_general(wr2_ref[...], r, (((0,), (0,)), ((), ())),
                              preferred_element_type=jnp.float32) + br2_ref[...]
        mean = out[0:1, :]
        ls = jnp.maximum(out[1:2, :], -3.0)

        if k == 0:
            @pl.when(zb == 0)
            def _():
                slb_s[...] = sxr_ref[0] * jnp.exp(ls) + mean - 2.0
                slb_ref[0] = slb_s[...]

        slb = slb_s[...]
        dev = (slb - mean + 2.0) * jnp.exp(-ls)
        logq = -0.5 * jnp.sum(LOG2PI + dev * dev) - jnp.sum(ls)
        logqs.append(jnp.full((1, 1, 1, 128), logq, jnp.float32))
    logq_ref[...] = jnp.concatenate(logqs, axis=1)


def kernel(node_features, parent_index, samp_z, samp_x_raw, W1, b1, W2, b2,
           Wr1, br1, Wr2, br2):
    f32 = jnp.float32
    pi = parent_index.astype(jnp.int32).reshape(B, 1, NDIM)
    sxr = samp_x_raw.reshape(B, 1, NDIM)
    b1r = b1.reshape(1, HID)
    b2r = b2.reshape(1, HID)
    wr1a = Wr1[:HID]
    wr1b = Wr1[HID:]
    br1c = br1.reshape(HID, 1)
    br2c = br2.reshape(2, 1)

    base = pl.pallas_call(
        _front_kernel,
        grid=(B,),
        in_specs=[
            pl.BlockSpec((1, NNODE, NTIPS), lambda b: (b, 0, 0)),
            pl.BlockSpec((1, 1, NDIM), lambda b: (b, 0, 0)),
            pl.BlockSpec((NTIPS, HID), lambda b: (0, 0)),
            pl.BlockSpec((1, HID), lambda b: (0, 0)),
            pl.BlockSpec((HID, HID), lambda b: (0, 0)),
            pl.BlockSpec((1, HID), lambda b: (0, 0)),
            pl.BlockSpec((HID, HID), lambda b: (0, 0)),
            pl.BlockSpec((HID, 1), lambda b: (0, 0)),
        ],
        out_specs=pl.BlockSpec((1, HID, NDIM), lambda b: (b, 0, 0)),
        out_shape=jax.ShapeDtypeStruct((B, HID, NDIM), f32),
    )(node_features, pi, W1, b1r, W2, b2r, wr1a, br1c)

    slb_p, logq_p = pl.pallas_call(
        _readout_kernel,
        grid=(B, Z // ZB),
        in_specs=[
            pl.BlockSpec((1, HID, NDIM), lambda b, z: (b, 0, 0)),
            pl.BlockSpec((1, ZB, NDIM, LAT), lambda b, z: (b, z, 0, 0)),
            pl.BlockSpec((LAT, HID), lambda b, z: (0, 0)),
            pl.BlockSpec((HID, 2), lambda b, z: (0, 0)),
            pl.BlockSpec((2, 1), lambda b, z: (0, 0)),
            pl.BlockSpec((1, 1, NDIM), lambda b, z: (b, 0, 0)),
        ],
        out_specs=[
            pl.BlockSpec((1, 1, NDIM), lambda b, z: (b, 0, 0)),
            pl.BlockSpec((1, ZB, 1, 128), lambda b, z: (b, z, 0, 0)),
        ],
        out_shape=[
            jax.ShapeDtypeStruct((B, 1, NDIM), f32),
            jax.ShapeDtypeStruct((B, Z, 1, 128), f32),
        ],
        scratch_shapes=[pltpu.VMEM((1, NDIM), f32)],
    )(base, samp_z, wr1b, Wr2, br2c, sxr)

    samp_log_branch = slb_p[:, 0, :]
    logq_branch_batch = logq_p[:, :, 0, 0]
    return (samp_log_branch, logq_branch_batch)
